# Initial kernel scaffold; baseline (speedup 1.0000x reference)
import jax
import jax.numpy as jnp
from jax.experimental import pallas as pl


def kernel(h, coord, edge_index, edge_attr, params):
    raise NotImplementedError("write your pallas kernel here")


# speedup vs baseline: 1.9039x; 1.9039x over previous
import functools
import jax
import jax.numpy as jnp
from jax import lax
from jax.experimental import pallas as pl
from jax.experimental.pallas import tpu as pltpu
from jax.experimental.pallas import tpu_sc as plsc

_N = 10000
_E = 320000
_NPAD = 10240
_EPAD = 327680
_D = 80
_HID = 64
_F32 = jnp.float32

_NW = 32
_PERW = _EPAD // _NW
_CH = 1024
_KSUB = _CH // 128
_NCHUNK = _PERW // _CH
_CHS = 512
_KSUBS = _CHS // 128
_NCHUNKS = _PERW // _CHS


def _silu(x):
    return x * jax.nn.sigmoid(x)


def _pack_body(h_ref, c_ref, w_ref, b_ref, out_ref):
    hh = jnp.dot(h_ref[...], w_ref[...], preferred_element_type=_F32) + b_ref[...]
    out_ref[...] = jnp.concatenate([hh, c_ref[...]], axis=1)


def _edge_body(g1_ref, g2_ref, ea_ref, w0a, w0b, w0r, w0e, b0, w1, b1,
               wc0, bc0, wc1, out_ref):
    g1 = g1_ref[...]
    g2 = g2_ref[...]
    h1 = g1[:, :_HID]
    h2 = g2[:, :_HID]
    cd = g1[:, _HID:] - g2[:, _HID:]
    radial = jnp.sum(cd * cd, axis=1, keepdims=True)
    m = (jnp.dot(h1, w0a[...], preferred_element_type=_F32)
         + jnp.dot(h2, w0b[...], preferred_element_type=_F32)
         + radial * w0r[...]
         + jnp.dot(ea_ref[...], w0e[...], preferred_element_type=_F32)
         + b0[...])
    m = _silu(m)
    ef = _silu(jnp.dot(m, w1[...], preferred_element_type=_F32) + b1[...])
    ch = _silu(jnp.dot(ef, wc0[...], preferred_element_type=_F32) + bc0[...])
    cm = jnp.sum(ch * wc1[...], axis=1, keepdims=True)
    tr = cd * cm
    lane = lax.broadcasted_iota(jnp.int32, tr.shape, 1)
    tr = jnp.where(lane == 15, 1.0, tr)
    out_ref[...] = jnp.concatenate([ef, tr], axis=1)


def _node_body(t_ref, a0_ref, a1_ref, wna, wnb, bn0, wn1, bn1, out_ref):
    t = t_ref[...]
    h = t[:, :_HID]
    c16 = t[:, _HID:]
    acc = a0_ref[...] + a1_ref[...]
    agg = acc[:, :_HID]
    s16 = acc[:, _HID:]
    cnt = jnp.maximum(acc[:, _D - 1:_D], 1.0)
    lane = lax.broadcasted_iota(jnp.int32, s16.shape, 1)
    cnew = jnp.where(lane < 3, c16 + s16 / cnt, 0.0)
    u = _silu(jnp.dot(h, wna[...], preferred_element_type=_F32)
              + jnp.dot(agg, wnb[...], preferred_element_type=_F32)
              + bn0[...])
    hn = h + jnp.dot(u, wn1[...], preferred_element_type=_F32) + bn1[...]
    out_ref[...] = jnp.concatenate([hn, cnew], axis=1)


def _final_body(t_ref, we, be, wo, bo, out_ref):
    h = t_ref[...][:, :_HID]
    z = jnp.dot(h, we[...], preferred_element_type=_F32) + be[...]
    out_ref[...] = jnp.dot(z, wo[...], preferred_element_type=_F32) + bo[...]


def _row_spec(bn, width):
    return pl.BlockSpec((bn, width), lambda i: (i, 0))


def _w_spec(shape):
    return pl.BlockSpec(shape, lambda i: (0,) * len(shape))


def _pack_call(h_pad, c16, w, b):
    bn = 1024
    return pl.pallas_call(
        _pack_body,
        grid=(_NPAD // bn,),
        in_specs=[_row_spec(bn, 128), _row_spec(bn, 16),
                  _w_spec((128, _HID)), _w_spec((1, _HID))],
        out_specs=_row_spec(bn, _D),
        out_shape=jax.ShapeDtypeStruct((_NPAD, _D), _F32),
    )(h_pad, c16, w, b)


def _edge_call(grow, gcol, ea, ws):
    be = 2048
    wspecs = [_w_spec((_HID, _HID)), _w_spec((_HID, _HID)), _w_spec((1, _HID)),
              _w_spec((8, _HID)), _w_spec((1, _HID)), _w_spec((_HID, _HID)),
              _w_spec((1, _HID)), _w_spec((_HID, _HID)), _w_spec((1, _HID)),
              _w_spec((1, _HID))]
    return pl.pallas_call(
        _edge_body,
        grid=(_EPAD // be,),
        in_specs=[_row_spec(be, _D), _row_spec(be, _D), _row_spec(be, 8)]
                 + wspecs,
        out_specs=_row_spec(be, _D),
        out_shape=jax.ShapeDtypeStruct((_EPAD, _D), _F32),
    )(grow, gcol, ea, *ws)


def _node_call(table, a0, a1, ws):
    bn = 1024
    wspecs = [_w_spec((_HID, _HID)), _w_spec((_HID, _HID)), _w_spec((1, _HID)),
              _w_spec((_HID, _HID)), _w_spec((1, _HID))]
    return pl.pallas_call(
        _node_body,
        grid=(_NPAD // bn,),
        in_specs=[_row_spec(bn, _D), _row_spec(bn, _D), _row_spec(bn, _D)]
                 + wspecs,
        out_specs=_row_spec(bn, _D),
        out_shape=jax.ShapeDtypeStruct((_NPAD, _D), _F32),
    )(table, a0, a1, *ws)


def _final_call(table, we, be, wo, bo):
    bn = 1024
    return pl.pallas_call(
        _final_body,
        grid=(_NPAD // bn,),
        in_specs=[_row_spec(bn, _D), _w_spec((_HID, _HID)), _w_spec((1, _HID)),
                  _w_spec((_HID, 8)), _w_spec((1, 8))],
        out_specs=_row_spec(bn, 8),
        out_shape=jax.ShapeDtypeStruct((_NPAD, 8), _F32),
    )(table, we, be, wo, bo)


def _gather_kernel():
    mesh = plsc.VectorSubcoreMesh(core_axis_name="c", subcore_axis_name="s")

    @functools.partial(
        pl.kernel,
        mesh=mesh,
        out_type=(jax.ShapeDtypeStruct((_EPAD, _D), _F32),
                  jax.ShapeDtypeStruct((_EPAD, _D), _F32)),
        scratch_types=[pltpu.VMEM((_KSUB, 128), jnp.int32),
                       pltpu.VMEM((_CH, _D), _F32),
                       pltpu.SemaphoreType.DMA],
        compiler_params=pltpu.CompilerParams(use_tc_tiling_on_sc=False),
    )
    def gk(tab_hbm, rows_hbm, cols_hbm, orow_hbm, ocol_hbm, idx_v, buf_v, sem):
        wid = lax.axis_index("s") * 2 + lax.axis_index("c")
        crow0 = wid * (_PERW // 128)
        ebase0 = wid * _PERW

        def one_gather(idx_hbm, crow, ebase, out_hbm):
            pltpu.sync_copy(idx_hbm.at[pl.ds(crow, _KSUB)], idx_v)
            cps = [pltpu.async_copy(tab_hbm.at[idx_v.at[j]],
                                    buf_v.at[pl.ds(j * 128, 128)], sem)
                   for j in range(_KSUB)]
            for c in cps:
                c.wait()
            pltpu.sync_copy(buf_v, out_hbm.at[pl.ds(ebase, _CH)])

        def body(i, carry):
            crow = crow0 + i * _KSUB
            ebase = ebase0 + i * _CH
            one_gather(rows_hbm, crow, ebase, orow_hbm)
            one_gather(cols_hbm, crow, ebase, ocol_hbm)
            return carry

        lax.fori_loop(0, _NCHUNK, body, 0)

    return gk


def _scatter_kernel():
    mesh = plsc.VectorSubcoreMesh(core_axis_name="c", subcore_axis_name="s")

    @functools.partial(
        pl.kernel,
        mesh=mesh,
        out_type=(jax.ShapeDtypeStruct((_NPAD, _D), _F32),
                  jax.ShapeDtypeStruct((_NPAD, _D), _F32)),
        scratch_types=[pltpu.VMEM((_KSUBS, 128), jnp.int32),
                       pltpu.VMEM((_CHS, _D), _F32),
                       pltpu.VMEM_SHARED((_NPAD, _D), _F32),
                       pltpu.SemaphoreType.DMA],
        compiler_params=pltpu.CompilerParams(use_tc_tiling_on_sc=False),
    )
    def sk(rows_hbm, vals_hbm, zeros_hbm, out0_hbm, out1_hbm,
           idx_v, buf_v, acc_sh, sem):
        cid = lax.axis_index("c")
        sid = lax.axis_index("s")
        wid = sid * 2 + cid
        nper = _NPAD // 16

        pltpu.sync_copy(zeros_hbm.at[pl.ds(sid * nper, nper)],
                        acc_sh.at[pl.ds(sid * nper, nper)])
        plsc.subcore_barrier()

        def body(i, carry):
            crow = wid * (_PERW // 128) + i * _KSUBS
            ebase = wid * _PERW + i * _CHS
            pltpu.sync_copy(rows_hbm.at[pl.ds(crow, _KSUBS)], idx_v)
            pltpu.sync_copy(vals_hbm.at[pl.ds(ebase, _CHS)], buf_v)
            for j in range(_KSUBS):
                pltpu.sync_copy(buf_v.at[pl.ds(j * 128, 128)],
                                acc_sh.at[idx_v.at[j]], add=True)
            return carry

        lax.fori_loop(0, _NCHUNKS, body, 0)
        plsc.subcore_barrier()

        src = acc_sh.at[pl.ds(sid * nper, nper)]

        @pl.when(cid == 0)
        def _():
            pltpu.sync_copy(src, out0_hbm.at[pl.ds(sid * nper, nper)])

        @pl.when(cid == 1)
        def _():
            pltpu.sync_copy(src, out1_hbm.at[pl.ds(sid * nper, nper)])

    return sk


def _prep_layer(lp):
    w0 = lp["edge0"]["W"]
    w0e = jnp.zeros((8, _HID), _F32).at[:4].set(w0[129:133])
    edge_ws = (w0[:64], w0[64:128], w0[128:129], w0e,
               lp["edge0"]["b"][None, :], lp["edge1"]["W"],
               lp["edge1"]["b"][None, :], lp["coord0"]["W"],
               lp["coord0"]["b"][None, :], lp["coord1"]["W"].reshape(1, _HID))
    wn = lp["node0"]["W"]
    node_ws = (wn[:64], wn[64:128], lp["node0"]["b"][None, :],
               lp["node1"]["W"], lp["node1"]["b"][None, :])
    return edge_ws, node_ws


def kernel(h, coord, edge_index, edge_attr, params):
    row = edge_index[0]
    col = edge_index[1]
    padi = jnp.full((_EPAD - _E,), _NPAD - 1, jnp.int32)
    rows2d = jnp.concatenate([row, padi]).reshape(_EPAD // 128, 128)
    cols2d = jnp.concatenate([col, padi]).reshape(_EPAD // 128, 128)
    ea = jnp.zeros((_EPAD, 8), _F32).at[:_E, :4].set(edge_attr)
    h_pad = jnp.zeros((_NPAD, 128), _F32).at[:_N].set(h)
    c16 = jnp.zeros((_NPAD, 16), _F32).at[:_N, :3].set(coord)
    zeros_tab = jnp.zeros((_NPAD, _D), _F32)

    gather = _gather_kernel()
    scatter = _scatter_kernel()

    table = _pack_call(h_pad, c16, params["emb_in"]["W"],
                       params["emb_in"]["b"][None, :])
    for lp in params["layers"]:
        edge_ws, node_ws = _prep_layer(lp)
        grow, gcol = gather(table, rows2d, cols2d)
        vals = _edge_call(grow, gcol, ea, edge_ws)
        a0, a1 = scatter(rows2d, vals, zeros_tab)
        table = _node_call(table, a0, a1, node_ws)

    wo = jnp.zeros((_HID, 8), _F32).at[:, 0].set(params["out_layer"]["W"][:, 0])
    bo = jnp.zeros((1, 8), _F32).at[0, 0].set(params["out_layer"]["b"][0])
    y = _final_call(table, params["emb_out"]["W"],
                    params["emb_out"]["b"][None, :], wo, bo)
    return y[:_N, :1]


# speedup vs baseline: 1.9713x; 1.0354x over previous
import functools
import jax
import jax.numpy as jnp
from jax import lax
from jax.experimental import pallas as pl
from jax.experimental.pallas import tpu as pltpu
from jax.experimental.pallas import tpu_sc as plsc

_N = 10000
_E = 320000
_NPAD = 10240
_EPAD = 327680
_D = 80
_HID = 64
_F32 = jnp.float32

_NW = 32
_PERW = _EPAD // _NW
_IDXROWS = _PERW // 128
_CH = 512
_KSUB = _CH // 128
_NCHUNK = _PERW // _CH
_CHS = 256
_KSUBS = _CHS // 128
_NCHUNKS = _PERW // _CHS


def _silu(x):
    return x * jax.nn.sigmoid(x)


def _pack_body(h_ref, c_ref, w_ref, b_ref, out_ref):
    hh = jnp.dot(h_ref[...], w_ref[...], preferred_element_type=_F32) + b_ref[...]
    out_ref[...] = jnp.concatenate([hh, c_ref[...]], axis=1)


def _edge_body(g1_ref, g2_ref, ea_ref, w0a, w0b, w0r, w0e, b0, w1, b1,
               wc0, bc0, wc1, out_ref):
    g1 = g1_ref[...]
    g2 = g2_ref[...]
    h1 = g1[:, :_HID]
    h2 = g2[:, :_HID]
    cd = g1[:, _HID:] - g2[:, _HID:]
    radial = jnp.sum(cd * cd, axis=1, keepdims=True)
    m = (jnp.dot(h1, w0a[...], preferred_element_type=_F32)
         + jnp.dot(h2, w0b[...], preferred_element_type=_F32)
         + radial * w0r[...]
         + jnp.dot(ea_ref[...], w0e[...], preferred_element_type=_F32)
         + b0[...])
    m = _silu(m)
    ef = _silu(jnp.dot(m, w1[...], preferred_element_type=_F32) + b1[...])
    ch = _silu(jnp.dot(ef, wc0[...], preferred_element_type=_F32) + bc0[...])
    cm = jnp.sum(ch * wc1[...], axis=1, keepdims=True)
    tr = cd * cm
    lane = lax.broadcasted_iota(jnp.int32, tr.shape, 1)
    tr = jnp.where(lane == 15, 1.0, tr)
    out_ref[...] = jnp.concatenate([ef, tr], axis=1)


def _node_body(t_ref, a0_ref, a1_ref, wna, wnb, bn0, wn1, bn1, out_ref):
    t = t_ref[...]
    h = t[:, :_HID]
    c16 = t[:, _HID:]
    acc = a0_ref[...] + a1_ref[...]
    agg = acc[:, :_HID]
    s16 = acc[:, _HID:]
    cnt = jnp.maximum(acc[:, _D - 1:_D], 1.0)
    lane = lax.broadcasted_iota(jnp.int32, s16.shape, 1)
    cnew = jnp.where(lane < 3, c16 + s16 / cnt, 0.0)
    u = _silu(jnp.dot(h, wna[...], preferred_element_type=_F32)
              + jnp.dot(agg, wnb[...], preferred_element_type=_F32)
              + bn0[...])
    hn = h + jnp.dot(u, wn1[...], preferred_element_type=_F32) + bn1[...]
    out_ref[...] = jnp.concatenate([hn, cnew], axis=1)


def _final_body(t_ref, we, be, wo, bo, out_ref):
    h = t_ref[...][:, :_HID]
    z = jnp.dot(h, we[...], preferred_element_type=_F32) + be[...]
    out_ref[...] = jnp.dot(z, wo[...], preferred_element_type=_F32) + bo[...]


def _row_spec(bn, width):
    return pl.BlockSpec((bn, width), lambda i: (i, 0))


def _w_spec(shape):
    return pl.BlockSpec(shape, lambda i: (0,) * len(shape))


def _pack_call(h_pad, c16, w, b):
    bn = 1024
    return pl.pallas_call(
        _pack_body,
        grid=(_NPAD // bn,),
        in_specs=[_row_spec(bn, 128), _row_spec(bn, 16),
                  _w_spec((128, _HID)), _w_spec((1, _HID))],
        out_specs=_row_spec(bn, _D),
        out_shape=jax.ShapeDtypeStruct((_NPAD, _D), _F32),
    )(h_pad, c16, w, b)


def _edge_call(grow, gcol, ea, ws):
    be = 2048
    wspecs = [_w_spec((_HID, _HID)), _w_spec((_HID, _HID)), _w_spec((1, _HID)),
              _w_spec((8, _HID)), _w_spec((1, _HID)), _w_spec((_HID, _HID)),
              _w_spec((1, _HID)), _w_spec((_HID, _HID)), _w_spec((1, _HID)),
              _w_spec((1, _HID))]
    return pl.pallas_call(
        _edge_body,
        grid=(_EPAD // be,),
        in_specs=[_row_spec(be, _D), _row_spec(be, _D), _row_spec(be, 8)]
                 + wspecs,
        out_specs=_row_spec(be, _D),
        out_shape=jax.ShapeDtypeStruct((_EPAD, _D), _F32),
    )(grow, gcol, ea, *ws)


def _node_call(table, a0, a1, ws):
    bn = 1024
    wspecs = [_w_spec((_HID, _HID)), _w_spec((_HID, _HID)), _w_spec((1, _HID)),
              _w_spec((_HID, _HID)), _w_spec((1, _HID))]
    return pl.pallas_call(
        _node_body,
        grid=(_NPAD // bn,),
        in_specs=[_row_spec(bn, _D), _row_spec(bn, _D), _row_spec(bn, _D)]
                 + wspecs,
        out_specs=_row_spec(bn, _D),
        out_shape=jax.ShapeDtypeStruct((_NPAD, _D), _F32),
    )(table, a0, a1, *ws)


def _final_call(table, we, be, wo, bo):
    bn = 1024
    return pl.pallas_call(
        _final_body,
        grid=(_NPAD // bn,),
        in_specs=[_row_spec(bn, _D), _w_spec((_HID, _HID)), _w_spec((1, _HID)),
                  _w_spec((_HID, 8)), _w_spec((1, 8))],
        out_specs=_row_spec(bn, 8),
        out_shape=jax.ShapeDtypeStruct((_NPAD, 8), _F32),
    )(table, we, be, wo, bo)


def _gather_kernel():
    mesh = plsc.VectorSubcoreMesh(core_axis_name="c", subcore_axis_name="s")

    @functools.partial(
        pl.kernel,
        mesh=mesh,
        out_type=(jax.ShapeDtypeStruct((_EPAD, _D), _F32),
                  jax.ShapeDtypeStruct((_EPAD, _D), _F32)),
        scratch_types=[pltpu.VMEM((_IDXROWS, 128), jnp.int32),
                       pltpu.VMEM((_IDXROWS, 128), jnp.int32),
                       pltpu.VMEM((2, _CH, _D), _F32),
                       pltpu.SemaphoreType.DMA,
                       pltpu.SemaphoreType.DMA,
                       pltpu.SemaphoreType.DMA,
                       pltpu.SemaphoreType.DMA],
        compiler_params=pltpu.CompilerParams(use_tc_tiling_on_sc=False),
    )
    def gk(tab_hbm, rows_hbm, cols_hbm, orow_hbm, ocol_hbm,
           ir_v, ic_v, buf_v, isem, gsem, wsem0, wsem1):
        wid = lax.axis_index("s") * 2 + lax.axis_index("c")
        crow0 = wid * _IDXROWS
        ebase0 = wid * _PERW
        wsems = (wsem0, wsem1)

        c1 = pltpu.async_copy(rows_hbm.at[pl.ds(crow0, _IDXROWS)], ir_v, isem)
        c2 = pltpu.async_copy(cols_hbm.at[pl.ds(crow0, _IDXROWS)], ic_v, isem)
        c1.wait()
        c2.wait()

        def body(i, carry):
            ebase = ebase0 + i * _CH
            for p, (idx_all, out_hbm) in enumerate(((ir_v, orow_hbm),
                                                    (ic_v, ocol_hbm))):
                @pl.when(i > 0)
                def _():
                    pltpu.make_async_copy(
                        buf_v.at[p], out_hbm.at[pl.ds(ebase0, _CH)],
                        wsems[p]).wait()
                cps = [pltpu.async_copy(
                    tab_hbm.at[idx_all.at[i * _KSUB + j]],
                    buf_v.at[p, pl.ds(j * 128, 128)], gsem)
                    for j in range(_KSUB)]
                for c in cps:
                    c.wait()
                pltpu.async_copy(buf_v.at[p], out_hbm.at[pl.ds(ebase, _CH)],
                                 wsems[p])
            return carry

        lax.fori_loop(0, _NCHUNK, body, 0)
        for p, out_hbm in ((0, orow_hbm), (1, ocol_hbm)):
            pltpu.make_async_copy(buf_v.at[p], out_hbm.at[pl.ds(ebase0, _CH)],
                                  wsems[p]).wait()

    return gk


def _scatter_kernel():
    mesh = plsc.VectorSubcoreMesh(core_axis_name="c", subcore_axis_name="s")

    @functools.partial(
        pl.kernel,
        mesh=mesh,
        out_type=(jax.ShapeDtypeStruct((_NPAD, _D), _F32),
                  jax.ShapeDtypeStruct((_NPAD, _D), _F32)),
        scratch_types=[pltpu.VMEM((_IDXROWS, 128), jnp.int32),
                       pltpu.VMEM((2, _CHS, _D), _F32),
                       pltpu.VMEM_SHARED((_NPAD, _D), _F32),
                       pltpu.SemaphoreType.DMA,
                       pltpu.SemaphoreType.DMA,
                       pltpu.SemaphoreType.DMA,
                       pltpu.SemaphoreType.DMA],
        compiler_params=pltpu.CompilerParams(use_tc_tiling_on_sc=False),
    )
    def sk(rows_hbm, vals_hbm, zeros_hbm, out0_hbm, out1_hbm,
           idx_v, buf_v, acc_sh, isem, asem, vsem0, vsem1):
        cid = lax.axis_index("c")
        sid = lax.axis_index("s")
        wid = sid * 2 + cid
        nper = _NPAD // 16
        ebase0 = wid * _PERW
        vsems = (vsem0, vsem1)

        ci = pltpu.async_copy(rows_hbm.at[pl.ds(wid * _IDXROWS, _IDXROWS)],
                              idx_v, isem)
        pltpu.sync_copy(zeros_hbm.at[pl.ds(sid * nper, nper)],
                        acc_sh.at[pl.ds(sid * nper, nper)])
        ci.wait()
        plsc.subcore_barrier()

        for b in range(2):
            pltpu.async_copy(vals_hbm.at[pl.ds(ebase0 + b * _CHS, _CHS)],
                             buf_v.at[b], vsems[b])

        def body(i, carry):
            for b in range(2):
                c = 2 * i + b
                ebase = ebase0 + c * _CHS
                pltpu.make_async_copy(vals_hbm.at[pl.ds(ebase0, _CHS)],
                                      buf_v.at[b], vsems[b]).wait()
                cps = [pltpu.async_copy(
                    buf_v.at[b, pl.ds(j * 128, 128)],
                    acc_sh.at[idx_v.at[c * _KSUBS + j]], asem, add=True)
                    for j in range(_KSUBS)]
                for cp in cps:
                    cp.wait()

                @pl.when(c + 2 < _NCHUNKS)
                def _():
                    pltpu.async_copy(
                        vals_hbm.at[pl.ds(ebase + 2 * _CHS, _CHS)],
                        buf_v.at[b], vsems[b])
            return carry

        lax.fori_loop(0, _NCHUNKS // 2, body, 0)
        plsc.subcore_barrier()

        src = acc_sh.at[pl.ds(sid * nper, nper)]

        @pl.when(cid == 0)
        def _():
            pltpu.sync_copy(src, out0_hbm.at[pl.ds(sid * nper, nper)])

        @pl.when(cid == 1)
        def _():
            pltpu.sync_copy(src, out1_hbm.at[pl.ds(sid * nper, nper)])

    return sk


def _prep_layer(lp):
    w0 = lp["edge0"]["W"]
    w0e = jnp.zeros((8, _HID), _F32).at[:4].set(w0[129:133])
    edge_ws = (w0[:64], w0[64:128], w0[128:129], w0e,
               lp["edge0"]["b"][None, :], lp["edge1"]["W"],
               lp["edge1"]["b"][None, :], lp["coord0"]["W"],
               lp["coord0"]["b"][None, :], lp["coord1"]["W"].reshape(1, _HID))
    wn = lp["node0"]["W"]
    node_ws = (wn[:64], wn[64:128], lp["node0"]["b"][None, :],
               lp["node1"]["W"], lp["node1"]["b"][None, :])
    return edge_ws, node_ws


def kernel(h, coord, edge_index, edge_attr, params):
    row = edge_index[0]
    col = edge_index[1]
    padi = jnp.full((_EPAD - _E,), _NPAD - 1, jnp.int32)
    rows2d = jnp.concatenate([row, padi]).reshape(_EPAD // 128, 128)
    cols2d = jnp.concatenate([col, padi]).reshape(_EPAD // 128, 128)
    ea = jnp.zeros((_EPAD, 8), _F32).at[:_E, :4].set(edge_attr)
    h_pad = jnp.zeros((_NPAD, 128), _F32).at[:_N].set(h)
    c16 = jnp.zeros((_NPAD, 16), _F32).at[:_N, :3].set(coord)
    zeros_tab = jnp.zeros((_NPAD, _D), _F32)

    gather = _gather_kernel()
    scatter = _scatter_kernel()

    table = _pack_call(h_pad, c16, params["emb_in"]["W"],
                       params["emb_in"]["b"][None, :])
    for lp in params["layers"]:
        edge_ws, node_ws = _prep_layer(lp)
        grow, gcol = gather(table, rows2d, cols2d)
        vals = _edge_call(grow, gcol, ea, edge_ws)
        a0, a1 = scatter(rows2d, vals, zeros_tab)
        table = _node_call(table, a0, a1, node_ws)

    wo = jnp.zeros((_HID, 8), _F32).at[:, 0].set(params["out_layer"]["W"][:, 0])
    bo = jnp.zeros((1, 8), _F32).at[0, 0].set(params["out_layer"]["b"][0])
    y = _final_call(table, params["emb_out"]["W"],
                    params["emb_out"]["b"][None, :], wo, bo)
    return y[:_N, :1]


# speedup vs baseline: 2.0854x; 1.0579x over previous
import functools
import jax
import jax.numpy as jnp
from jax import lax
from jax.experimental import pallas as pl
from jax.experimental.pallas import tpu as pltpu
from jax.experimental.pallas import tpu_sc as plsc

_N = 10000
_E = 320000
_NPAD = 10240
_EPAD = 327680
_D = 80
_HID = 64
_F32 = jnp.float32

_NW = 32
_PERW = _EPAD // _NW
_IDXROWS = _PERW // 128
_CH = 512
_KSUB = _CH // 128
_NCHUNK = _PERW // _CH
_CHS = 256
_KSUBS = _CHS // 128
_NCHUNKS = _PERW // _CHS


def _silu(x):
    return x * jax.nn.sigmoid(x)


def _pack_body(h_ref, c_ref, w_ref, b_ref, out_ref):
    hh = jnp.dot(h_ref[...], w_ref[...], preferred_element_type=_F32) + b_ref[...]
    out_ref[...] = jnp.concatenate([hh, c_ref[...]], axis=1)


def _edge_body(g1_ref, g2_ref, ea_ref, w0a, w0b, w0r, w0e, b0, w1, b1,
               wc0, bc0, wc1, out_ref):
    g1 = g1_ref[...]
    g2 = g2_ref[...]
    h1 = g1[:, :_HID]
    h2 = g2[:, :_HID]
    cd = g1[:, _HID:] - g2[:, _HID:]
    radial = jnp.sum(cd * cd, axis=1, keepdims=True)
    m = (jnp.dot(h1, w0a[...], preferred_element_type=_F32)
         + jnp.dot(h2, w0b[...], preferred_element_type=_F32)
         + radial * w0r[...]
         + jnp.dot(ea_ref[...], w0e[...], preferred_element_type=_F32)
         + b0[...])
    m = _silu(m)
    ef = _silu(jnp.dot(m, w1[...], preferred_element_type=_F32) + b1[...])
    ch = _silu(jnp.dot(ef, wc0[...], preferred_element_type=_F32) + bc0[...])
    cm = jnp.sum(ch * wc1[...], axis=1, keepdims=True)
    tr = cd * cm
    lane = lax.broadcasted_iota(jnp.int32, tr.shape, 1)
    tr = jnp.where(lane == 15, 1.0, tr)
    out_ref[...] = jnp.concatenate([ef, tr], axis=1)


def _node_body(t_ref, a0_ref, a1_ref, a2_ref, a3_ref,
               wna, wnb, bn0, wn1, bn1, out_ref):
    t = t_ref[...]
    h = t[:, :_HID]
    c16 = t[:, _HID:]
    acc = (a0_ref[...] + a1_ref[...]) + (a2_ref[...] + a3_ref[...])
    agg = acc[:, :_HID]
    s16 = acc[:, _HID:]
    cnt = jnp.maximum(acc[:, _D - 1:_D], 1.0)
    lane = lax.broadcasted_iota(jnp.int32, s16.shape, 1)
    cnew = jnp.where(lane < 3, c16 + s16 / cnt, 0.0)
    u = _silu(jnp.dot(h, wna[...], preferred_element_type=_F32)
              + jnp.dot(agg, wnb[...], preferred_element_type=_F32)
              + bn0[...])
    hn = h + jnp.dot(u, wn1[...], preferred_element_type=_F32) + bn1[...]
    out_ref[...] = jnp.concatenate([hn, cnew], axis=1)


def _final_body(t_ref, we, be, wo, bo, out_ref):
    h = t_ref[...][:, :_HID]
    z = jnp.dot(h, we[...], preferred_element_type=_F32) + be[...]
    out_ref[...] = jnp.dot(z, wo[...], preferred_element_type=_F32) + bo[...]


def _row_spec(bn, width):
    return pl.BlockSpec((bn, width), lambda i: (i, 0))


def _w_spec(shape):
    return pl.BlockSpec(shape, lambda i: (0,) * len(shape))


def _pack_call(h_pad, c16, w, b):
    bn = 1024
    return pl.pallas_call(
        _pack_body,
        grid=(_NPAD // bn,),
        in_specs=[_row_spec(bn, 128), _row_spec(bn, 16),
                  _w_spec((128, _HID)), _w_spec((1, _HID))],
        out_specs=_row_spec(bn, _D),
        out_shape=jax.ShapeDtypeStruct((_NPAD, _D), _F32),
    )(h_pad, c16, w, b)


def _edge_call(grow, gcol, ea, ws):
    be = 2048
    ep = grow.shape[0]
    wspecs = [_w_spec((_HID, _HID)), _w_spec((_HID, _HID)), _w_spec((1, _HID)),
              _w_spec((8, _HID)), _w_spec((1, _HID)), _w_spec((_HID, _HID)),
              _w_spec((1, _HID)), _w_spec((_HID, _HID)), _w_spec((1, _HID)),
              _w_spec((1, _HID))]
    return pl.pallas_call(
        _edge_body,
        grid=(ep // be,),
        in_specs=[_row_spec(be, _D), _row_spec(be, _D), _row_spec(be, 8)]
                 + wspecs,
        out_specs=_row_spec(be, _D),
        out_shape=jax.ShapeDtypeStruct((ep, _D), _F32),
    )(grow, gcol, ea, *ws)


def _node_call(table, accs, ws):
    bn = 1024
    wspecs = [_w_spec((_HID, _HID)), _w_spec((_HID, _HID)), _w_spec((1, _HID)),
              _w_spec((_HID, _HID)), _w_spec((1, _HID))]
    return pl.pallas_call(
        _node_body,
        grid=(_NPAD // bn,),
        in_specs=[_row_spec(bn, _D)] * 5 + wspecs,
        out_specs=_row_spec(bn, _D),
        out_shape=jax.ShapeDtypeStruct((_NPAD, _D), _F32),
    )(table, *accs, *ws)


def _final_call(table, we, be, wo, bo):
    bn = 1024
    return pl.pallas_call(
        _final_body,
        grid=(_NPAD // bn,),
        in_specs=[_row_spec(bn, _D), _w_spec((_HID, _HID)), _w_spec((1, _HID)),
                  _w_spec((_HID, 8)), _w_spec((1, 8))],
        out_specs=_row_spec(bn, 8),
        out_shape=jax.ShapeDtypeStruct((_NPAD, 8), _F32),
    )(table, we, be, wo, bo)


def _gather_kernel(ep):
    perw = ep // _NW
    idxrows = perw // 128
    nchunk = perw // _CH
    mesh = plsc.VectorSubcoreMesh(core_axis_name="c", subcore_axis_name="s")

    @functools.partial(
        pl.kernel,
        mesh=mesh,
        out_type=(jax.ShapeDtypeStruct((ep, _D), _F32),
                  jax.ShapeDtypeStruct((ep, _D), _F32)),
        scratch_types=[pltpu.VMEM((idxrows, 128), jnp.int32),
                       pltpu.VMEM((idxrows, 128), jnp.int32),
                       pltpu.VMEM((2, _CH, _D), _F32),
                       pltpu.SemaphoreType.DMA,
                       pltpu.SemaphoreType.DMA,
                       pltpu.SemaphoreType.DMA,
                       pltpu.SemaphoreType.DMA],
        compiler_params=pltpu.CompilerParams(use_tc_tiling_on_sc=False),
    )
    def gk(tab_hbm, rows_hbm, cols_hbm, orow_hbm, ocol_hbm,
           ir_v, ic_v, buf_v, isem, gsem, wsem0, wsem1):
        wid = lax.axis_index("s") * 2 + lax.axis_index("c")
        crow0 = wid * idxrows
        ebase0 = wid * perw
        wsems = (wsem0, wsem1)

        c1 = pltpu.async_copy(rows_hbm.at[pl.ds(crow0, idxrows)], ir_v, isem)
        c2 = pltpu.async_copy(cols_hbm.at[pl.ds(crow0, idxrows)], ic_v, isem)
        c1.wait()
        c2.wait()

        def body(i, carry):
            ebase = ebase0 + i * _CH
            for p, (idx_all, out_hbm) in enumerate(((ir_v, orow_hbm),
                                                    (ic_v, ocol_hbm))):
                @pl.when(i > 0)
                def _():
                    pltpu.make_async_copy(
                        buf_v.at[p], out_hbm.at[pl.ds(ebase0, _CH)],
                        wsems[p]).wait()
                cps = [pltpu.async_copy(
                    tab_hbm.at[idx_all.at[i * _KSUB + j]],
                    buf_v.at[p, pl.ds(j * 128, 128)], gsem)
                    for j in range(_KSUB)]
                for c in cps:
                    c.wait()
                pltpu.async_copy(buf_v.at[p], out_hbm.at[pl.ds(ebase, _CH)],
                                 wsems[p])
            return carry

        lax.fori_loop(0, nchunk, body, 0)
        for p, out_hbm in ((0, orow_hbm), (1, ocol_hbm)):
            pltpu.make_async_copy(buf_v.at[p], out_hbm.at[pl.ds(ebase0, _CH)],
                                  wsems[p]).wait()

    return gk


def _scatter_kernel(ep):
    perw = ep // _NW
    idxrows = perw // 128
    nchunks = perw // _CHS
    mesh = plsc.VectorSubcoreMesh(core_axis_name="c", subcore_axis_name="s")

    @functools.partial(
        pl.kernel,
        mesh=mesh,
        out_type=(jax.ShapeDtypeStruct((_NPAD, _D), _F32),
                  jax.ShapeDtypeStruct((_NPAD, _D), _F32)),
        scratch_types=[pltpu.VMEM((idxrows, 128), jnp.int32),
                       pltpu.VMEM((2, _CHS, _D), _F32),
                       pltpu.VMEM_SHARED((_NPAD, _D), _F32),
                       pltpu.SemaphoreType.DMA,
                       pltpu.SemaphoreType.DMA,
                       pltpu.SemaphoreType.DMA,
                       pltpu.SemaphoreType.DMA],
        compiler_params=pltpu.CompilerParams(use_tc_tiling_on_sc=False),
    )
    def sk(rows_hbm, vals_hbm, zeros_hbm, out0_hbm, out1_hbm,
           idx_v, buf_v, acc_sh, isem, asem, vsem0, vsem1):
        cid = lax.axis_index("c")
        sid = lax.axis_index("s")
        wid = sid * 2 + cid
        nper = _NPAD // 16
        ebase0 = wid * perw
        vsems = (vsem0, vsem1)

        ci = pltpu.async_copy(rows_hbm.at[pl.ds(wid * idxrows, idxrows)],
                              idx_v, isem)
        pltpu.sync_copy(zeros_hbm.at[pl.ds(sid * nper, nper)],
                        acc_sh.at[pl.ds(sid * nper, nper)])
        ci.wait()
        plsc.subcore_barrier()

        for b in range(2):
            pltpu.async_copy(vals_hbm.at[pl.ds(ebase0 + b * _CHS, _CHS)],
                             buf_v.at[b], vsems[b])

        def body(i, carry):
            for b in range(2):
                c = 2 * i + b
                ebase = ebase0 + c * _CHS
                pltpu.make_async_copy(vals_hbm.at[pl.ds(ebase0, _CHS)],
                                      buf_v.at[b], vsems[b]).wait()
                cps = [pltpu.async_copy(
                    buf_v.at[b, pl.ds(j * 128, 128)],
                    acc_sh.at[idx_v.at[c * _KSUBS + j]], asem, add=True)
                    for j in range(_KSUBS)]
                for cp in cps:
                    cp.wait()

                @pl.when(c + 2 < nchunks)
                def _():
                    pltpu.async_copy(
                        vals_hbm.at[pl.ds(ebase + 2 * _CHS, _CHS)],
                        buf_v.at[b], vsems[b])
            return carry

        lax.fori_loop(0, nchunks // 2, body, 0)
        plsc.subcore_barrier()

        src = acc_sh.at[pl.ds(sid * nper, nper)]

        @pl.when(cid == 0)
        def _():
            pltpu.sync_copy(src, out0_hbm.at[pl.ds(sid * nper, nper)])

        @pl.when(cid == 1)
        def _():
            pltpu.sync_copy(src, out1_hbm.at[pl.ds(sid * nper, nper)])

    return sk


def _prep_layer(lp):
    w0 = lp["edge0"]["W"]
    w0e = jnp.zeros((8, _HID), _F32).at[:4].set(w0[129:133])
    edge_ws = (w0[:64], w0[64:128], w0[128:129], w0e,
               lp["edge0"]["b"][None, :], lp["edge1"]["W"],
               lp["edge1"]["b"][None, :], lp["coord0"]["W"],
               lp["coord0"]["b"][None, :], lp["coord1"]["W"].reshape(1, _HID))
    wn = lp["node0"]["W"]
    node_ws = (wn[:64], wn[64:128], lp["node0"]["b"][None, :],
               lp["node1"]["W"], lp["node1"]["b"][None, :])
    return edge_ws, node_ws


def kernel(h, coord, edge_index, edge_attr, params):
    row = edge_index[0]
    col = edge_index[1]
    eh = _EPAD // 2
    padi = jnp.full((_EPAD - _E,), _NPAD - 1, jnp.int32)
    rows2d = jnp.concatenate([row, padi]).reshape(_EPAD // 128, 128)
    cols2d = jnp.concatenate([col, padi]).reshape(_EPAD // 128, 128)
    rA, rB = rows2d[:eh // 128], rows2d[eh // 128:]
    cA, cB = cols2d[:eh // 128], cols2d[eh // 128:]
    ea = jnp.zeros((_EPAD, 8), _F32).at[:_E, :4].set(edge_attr)
    h_pad = jnp.zeros((_NPAD, 128), _F32).at[:_N].set(h)
    c16 = jnp.zeros((_NPAD, 16), _F32).at[:_N, :3].set(coord)
    zeros_tab = jnp.zeros((_NPAD, _D), _F32)

    gather = _gather_kernel(eh)
    scatter = _scatter_kernel(eh)
    eaA, eaB = ea[:eh], ea[eh:]

    table = _pack_call(h_pad, c16, params["emb_in"]["W"],
                       params["emb_in"]["b"][None, :])
    for lp in params["layers"]:
        edge_ws, node_ws = _prep_layer(lp)
        growA, gcolA = gather(table, rA, cA)
        valsA = _edge_call(growA, gcolA, eaA, edge_ws)
        growB, gcolB = gather(table, rB, cB)
        valsB = _edge_call(growB, gcolB, eaB, edge_ws)
        a0, a1 = scatter(rA, valsA, zeros_tab)
        a2, a3 = scatter(rB, valsB, zeros_tab)
        table = _node_call(table, (a0, a1, a2, a3), node_ws)

    wo = jnp.zeros((_HID, 8), _F32).at[:, 0].set(params["out_layer"]["W"][:, 0])
    bo = jnp.zeros((1, 8), _F32).at[0, 0].set(params["out_layer"]["b"][0])
    y = _final_call(table, params["emb_out"]["W"],
                    params["emb_out"]["b"][None, :], wo, bo)
    return y[:_N, :1]


# speedup vs baseline: 2.4155x; 1.1583x over previous
import functools
import jax
import jax.numpy as jnp
from jax import lax
from jax.experimental import pallas as pl
from jax.experimental.pallas import tpu as pltpu
from jax.experimental.pallas import tpu_sc as plsc

_N = 10000
_E = 320000
_NPAD = 10240
_EPAD = 327680
_HID = 64
_DC = 16
_F32 = jnp.float32

_NW = 32
_CH = 512
_KSUB = _CH // 128
_CHS = 256
_KSUBS = _CHS // 128


def _silu(x):
    return x * jax.nn.sigmoid(x)


def _pack_body(h_ref, w_ref, b_ref, out_ref):
    out_ref[...] = jnp.dot(h_ref[...], w_ref[...],
                           preferred_element_type=_F32) + b_ref[...]


def _edge_body(h1_ref, h2_ref, c1_ref, c2_ref, ea_ref,
               w0a, w0b, w0r, w0e, b0, w1, b1, wc0, bc0, wc1,
               ef_ref, tr_ref):
    cd = c1_ref[...] - c2_ref[...]
    radial = jnp.sum(cd * cd, axis=1, keepdims=True)
    m = (jnp.dot(h1_ref[...], w0a[...], preferred_element_type=_F32)
         + jnp.dot(h2_ref[...], w0b[...], preferred_element_type=_F32)
         + radial * w0r[...]
         + jnp.dot(ea_ref[...], w0e[...], preferred_element_type=_F32)
         + b0[...])
    m = _silu(m)
    ef = _silu(jnp.dot(m, w1[...], preferred_element_type=_F32) + b1[...])
    ch = _silu(jnp.dot(ef, wc0[...], preferred_element_type=_F32) + bc0[...])
    cm = jnp.sum(ch * wc1[...], axis=1, keepdims=True)
    tr = cd * cm
    lane = lax.broadcasted_iota(jnp.int32, tr.shape, 1)
    tr_ref[...] = jnp.where(lane == 15, 1.0, tr)
    ef_ref[...] = ef


def _node_body(th_ref, tc_ref, a0, a1, a2, a3, b0_, b1_, b2_, b3_,
               wna, wnb, bn0, wn1, bn1, oh_ref, oc_ref):
    h = th_ref[...]
    agg = (a0[...] + a1[...]) + (a2[...] + a3[...])
    s16 = (b0_[...] + b1_[...]) + (b2_[...] + b3_[...])
    cnt = jnp.maximum(s16[:, 15:16], 1.0)
    lane = lax.broadcasted_iota(jnp.int32, s16.shape, 1)
    oc_ref[...] = jnp.where(lane < 3, tc_ref[...] + s16 / cnt, 0.0)
    u = _silu(jnp.dot(h, wna[...], preferred_element_type=_F32)
              + jnp.dot(agg, wnb[...], preferred_element_type=_F32)
              + bn0[...])
    oh_ref[...] = h + jnp.dot(u, wn1[...], preferred_element_type=_F32) \
        + bn1[...]


def _final_body(th_ref, we, be, wo, bo, out_ref):
    z = jnp.dot(th_ref[...], we[...], preferred_element_type=_F32) + be[...]
    out_ref[...] = jnp.dot(z, wo[...], preferred_element_type=_F32) + bo[...]


def _row_spec(bn, width):
    return pl.BlockSpec((bn, width), lambda i: (i, 0))


def _w_spec(shape):
    return pl.BlockSpec(shape, lambda i: (0,) * len(shape))


def _pack_call(h_pad, w, b):
    bn = 1024
    return pl.pallas_call(
        _pack_body,
        grid=(_NPAD // bn,),
        in_specs=[_row_spec(bn, 128), _w_spec((128, _HID)), _w_spec((1, _HID))],
        out_specs=_row_spec(bn, _HID),
        out_shape=jax.ShapeDtypeStruct((_NPAD, _HID), _F32),
    )(h_pad, w, b)


def _edge_call(h1, h2, c1, c2, ea, ws):
    be = 2048
    ep = h1.shape[0]
    wspecs = [_w_spec((_HID, _HID)), _w_spec((_HID, _HID)), _w_spec((1, _HID)),
              _w_spec((8, _HID)), _w_spec((1, _HID)), _w_spec((_HID, _HID)),
              _w_spec((1, _HID)), _w_spec((_HID, _HID)), _w_spec((1, _HID)),
              _w_spec((1, _HID))]
    return pl.pallas_call(
        _edge_body,
        grid=(ep // be,),
        in_specs=[_row_spec(be, _HID), _row_spec(be, _HID),
                  _row_spec(be, _DC), _row_spec(be, _DC), _row_spec(be, 8)]
                 + wspecs,
        out_specs=(_row_spec(be, _HID), _row_spec(be, _DC)),
        out_shape=(jax.ShapeDtypeStruct((ep, _HID), _F32),
                   jax.ShapeDtypeStruct((ep, _DC), _F32)),
    )(h1, h2, c1, c2, ea, *ws)


def _node_call(tabh, tabc, accs64, accs16, ws):
    bn = 1024
    wspecs = [_w_spec((_HID, _HID)), _w_spec((_HID, _HID)), _w_spec((1, _HID)),
              _w_spec((_HID, _HID)), _w_spec((1, _HID))]
    return pl.pallas_call(
        _node_body,
        grid=(_NPAD // bn,),
        in_specs=[_row_spec(bn, _HID), _row_spec(bn, _DC)]
                 + [_row_spec(bn, _HID)] * 4 + [_row_spec(bn, _DC)] * 4
                 + wspecs,
        out_specs=(_row_spec(bn, _HID), _row_spec(bn, _DC)),
        out_shape=(jax.ShapeDtypeStruct((_NPAD, _HID), _F32),
                   jax.ShapeDtypeStruct((_NPAD, _DC), _F32)),
    )(tabh, tabc, *accs64, *accs16, *ws)


def _final_call(tabh, we, be, wo, bo):
    bn = 1024
    return pl.pallas_call(
        _final_body,
        grid=(_NPAD // bn,),
        in_specs=[_row_spec(bn, _HID), _w_spec((_HID, _HID)), _w_spec((1, _HID)),
                  _w_spec((_HID, 8)), _w_spec((1, 8))],
        out_specs=_row_spec(bn, 8),
        out_shape=jax.ShapeDtypeStruct((_NPAD, 8), _F32),
    )(tabh, we, be, wo, bo)


def _gather_kernel(ep):
    perw = ep // _NW
    idxrows = perw // 128
    nchunk = perw // _CH
    mesh = plsc.VectorSubcoreMesh(core_axis_name="c", subcore_axis_name="s")

    @functools.partial(
        pl.kernel,
        mesh=mesh,
        out_type=(jax.ShapeDtypeStruct((ep, _HID), _F32),
                  jax.ShapeDtypeStruct((ep, _HID), _F32),
                  jax.ShapeDtypeStruct((ep, _DC), _F32),
                  jax.ShapeDtypeStruct((ep, _DC), _F32)),
        scratch_types=[pltpu.VMEM((idxrows, 128), jnp.int32),
                       pltpu.VMEM((idxrows, 128), jnp.int32),
                       pltpu.VMEM((2, _CH, _HID), _F32),
                       pltpu.VMEM((2, _CH, _DC), _F32),
                       pltpu.SemaphoreType.DMA,
                       pltpu.SemaphoreType.DMA,
                       pltpu.SemaphoreType.DMA,
                       pltpu.SemaphoreType.DMA],
        compiler_params=pltpu.CompilerParams(use_tc_tiling_on_sc=False),
    )
    def gk(tabh_hbm, tabc_hbm, rows_hbm, cols_hbm,
           oh1_hbm, oh2_hbm, oc1_hbm, oc2_hbm,
           ir_v, ic_v, bufh_v, bufc_v, isem, gsem, wsem0, wsem1):
        wid = lax.axis_index("s") * 2 + lax.axis_index("c")
        crow0 = wid * idxrows
        ebase0 = wid * perw
        wsems = (wsem0, wsem1)

        c1 = pltpu.async_copy(rows_hbm.at[pl.ds(crow0, idxrows)], ir_v, isem)
        c2 = pltpu.async_copy(cols_hbm.at[pl.ds(crow0, idxrows)], ic_v, isem)
        c1.wait()
        c2.wait()

        def body(i, carry):
            ebase = ebase0 + i * _CH
            for p, (idx_all, oh_hbm, oc_hbm) in enumerate(
                    ((ir_v, oh1_hbm, oc1_hbm), (ic_v, oh2_hbm, oc2_hbm))):
                @pl.when(i > 0)
                def _():
                    pltpu.make_async_copy(
                        bufh_v.at[p], oh_hbm.at[pl.ds(ebase0, _CH)],
                        wsems[p]).wait()
                    pltpu.make_async_copy(
                        bufc_v.at[p], oc_hbm.at[pl.ds(ebase0, _CH)],
                        wsems[p]).wait()
                cps = []
                for j in range(_KSUB):
                    idx = idx_all.at[i * _KSUB + j]
                    sl = pl.ds(j * 128, 128)
                    cps.append(pltpu.async_copy(
                        tabh_hbm.at[idx], bufh_v.at[p, sl], gsem))
                    cps.append(pltpu.async_copy(
                        tabc_hbm.at[idx], bufc_v.at[p, sl], gsem))
                for c in cps:
                    c.wait()
                pltpu.async_copy(bufh_v.at[p], oh_hbm.at[pl.ds(ebase, _CH)],
                                 wsems[p])
                pltpu.async_copy(bufc_v.at[p], oc_hbm.at[pl.ds(ebase, _CH)],
                                 wsems[p])
            return carry

        lax.fori_loop(0, nchunk, body, 0)
        for p, oh_hbm, oc_hbm in ((0, oh1_hbm, oc1_hbm), (1, oh2_hbm, oc2_hbm)):
            pltpu.make_async_copy(bufh_v.at[p], oh_hbm.at[pl.ds(ebase0, _CH)],
                                  wsems[p]).wait()
            pltpu.make_async_copy(bufc_v.at[p], oc_hbm.at[pl.ds(ebase0, _CH)],
                                  wsems[p]).wait()

    return gk


def _scatter_kernel(ep):
    perw = ep // _NW
    idxrows = perw // 128
    nchunks = perw // _CHS
    mesh = plsc.VectorSubcoreMesh(core_axis_name="c", subcore_axis_name="s")

    @functools.partial(
        pl.kernel,
        mesh=mesh,
        out_type=(jax.ShapeDtypeStruct((_NPAD, _HID), _F32),
                  jax.ShapeDtypeStruct((_NPAD, _HID), _F32),
                  jax.ShapeDtypeStruct((_NPAD, _DC), _F32),
                  jax.ShapeDtypeStruct((_NPAD, _DC), _F32)),
        scratch_types=[pltpu.VMEM((idxrows, 128), jnp.int32),
                       pltpu.VMEM((2, _CHS, _HID), _F32),
                       pltpu.VMEM((2, _CHS, _DC), _F32),
                       pltpu.VMEM_SHARED((_NPAD, _HID), _F32),
                       pltpu.VMEM_SHARED((_NPAD, _DC), _F32),
                       pltpu.SemaphoreType.DMA,
                       pltpu.SemaphoreType.DMA,
                       pltpu.SemaphoreType.DMA,
                       pltpu.SemaphoreType.DMA],
        compiler_params=pltpu.CompilerParams(use_tc_tiling_on_sc=False),
    )
    def sk(rows_hbm, ef_hbm, tr_hbm, zh_hbm, zc_hbm,
           oh0_hbm, oh1_hbm, oc0_hbm, oc1_hbm,
           idx_v, bufh_v, bufc_v, acch_sh, accc_sh, isem, asem, vsem0, vsem1):
        cid = lax.axis_index("c")
        sid = lax.axis_index("s")
        wid = sid * 2 + cid
        nper = _NPAD // 16
        ebase0 = wid * perw
        vsems = (vsem0, vsem1)
        nsl = pl.ds(sid * nper, nper)

        ci = pltpu.async_copy(rows_hbm.at[pl.ds(wid * idxrows, idxrows)],
                              idx_v, isem)
        pltpu.sync_copy(zh_hbm.at[nsl], acch_sh.at[nsl])
        pltpu.sync_copy(zc_hbm.at[nsl], accc_sh.at[nsl])
        ci.wait()
        plsc.subcore_barrier()

        for b in range(2):
            sl = pl.ds(ebase0 + b * _CHS, _CHS)
            pltpu.async_copy(ef_hbm.at[sl], bufh_v.at[b], vsems[b])
            pltpu.async_copy(tr_hbm.at[sl], bufc_v.at[b], vsems[b])

        def body(i, carry):
            for b in range(2):
                c = 2 * i + b
                ebase = ebase0 + c * _CHS
                pltpu.make_async_copy(ef_hbm.at[pl.ds(ebase0, _CHS)],
                                      bufh_v.at[b], vsems[b]).wait()
                pltpu.make_async_copy(tr_hbm.at[pl.ds(ebase0, _CHS)],
                                      bufc_v.at[b], vsems[b]).wait()
                cps = []
                for j in range(_KSUBS):
                    idx = idx_v.at[c * _KSUBS + j]
                    sl = pl.ds(j * 128, 128)
                    cps.append(pltpu.async_copy(
                        bufh_v.at[b, sl], acch_sh.at[idx], asem, add=True))
                    cps.append(pltpu.async_copy(
                        bufc_v.at[b, sl], accc_sh.at[idx], asem, add=True))
                for cp in cps:
                    cp.wait()

                @pl.when(c + 2 < nchunks)
                def _():
                    sl2 = pl.ds(ebase + 2 * _CHS, _CHS)
                    pltpu.async_copy(ef_hbm.at[sl2], bufh_v.at[b], vsems[b])
                    pltpu.async_copy(tr_hbm.at[sl2], bufc_v.at[b], vsems[b])
            return carry

        lax.fori_loop(0, nchunks // 2, body, 0)
        plsc.subcore_barrier()

        @pl.when(cid == 0)
        def _():
            pltpu.sync_copy(acch_sh.at[nsl], oh0_hbm.at[nsl])
            pltpu.sync_copy(accc_sh.at[nsl], oc0_hbm.at[nsl])

        @pl.when(cid == 1)
        def _():
            pltpu.sync_copy(acch_sh.at[nsl], oh1_hbm.at[nsl])
            pltpu.sync_copy(accc_sh.at[nsl], oc1_hbm.at[nsl])

    return sk


def _prep_layer(lp):
    w0 = lp["edge0"]["W"]
    w0e = jnp.zeros((8, _HID), _F32).at[:4].set(w0[129:133])
    edge_ws = (w0[:64], w0[64:128], w0[128:129], w0e,
               lp["edge0"]["b"][None, :], lp["edge1"]["W"],
               lp["edge1"]["b"][None, :], lp["coord0"]["W"],
               lp["coord0"]["b"][None, :], lp["coord1"]["W"].reshape(1, _HID))
    wn = lp["node0"]["W"]
    node_ws = (wn[:64], wn[64:128], lp["node0"]["b"][None, :],
               lp["node1"]["W"], lp["node1"]["b"][None, :])
    return edge_ws, node_ws


def kernel(h, coord, edge_index, edge_attr, params):
    row = edge_index[0]
    col = edge_index[1]
    eh = _EPAD // 2
    padi = jnp.full((_EPAD - _E,), _NPAD - 1, jnp.int32)
    rows2d = jnp.concatenate([row, padi]).reshape(_EPAD // 128, 128)
    cols2d = jnp.concatenate([col, padi]).reshape(_EPAD // 128, 128)
    rA, rB = rows2d[:eh // 128], rows2d[eh // 128:]
    cA, cB = cols2d[:eh // 128], cols2d[eh // 128:]
    ea = jnp.zeros((_EPAD, 8), _F32).at[:_E, :4].set(edge_attr)
    eaA, eaB = ea[:eh], ea[eh:]
    h_pad = jnp.zeros((_NPAD, 128), _F32).at[:_N].set(h)
    tabc = jnp.zeros((_NPAD, _DC), _F32).at[:_N, :3].set(coord)
    zh = jnp.zeros((_NPAD, _HID), _F32)
    zc = jnp.zeros((_NPAD, _DC), _F32)

    gather = _gather_kernel(eh)
    scatter = _scatter_kernel(eh)

    tabh = _pack_call(h_pad, params["emb_in"]["W"],
                      params["emb_in"]["b"][None, :])
    for lp in params["layers"]:
        edge_ws, node_ws = _prep_layer(lp)
        h1A, h2A, c1A, c2A = gather(tabh, tabc, rA, cA)
        efA, trA = _edge_call(h1A, h2A, c1A, c2A, eaA, edge_ws)
        h1B, h2B, c1B, c2B = gather(tabh, tabc, rB, cB)
        efB, trB = _edge_call(h1B, h2B, c1B, c2B, eaB, edge_ws)
        ah0, ah1, ac0, ac1 = scatter(rA, efA, trA, zh, zc)
        bh0, bh1, bc0, bc1 = scatter(rB, efB, trB, zh, zc)
        tabh, tabc = _node_call(tabh, tabc, (ah0, ah1, bh0, bh1),
                                (ac0, ac1, bc0, bc1), node_ws)

    wo = jnp.zeros((_HID, 8), _F32).at[:, 0].set(params["out_layer"]["W"][:, 0])
    bo = jnp.zeros((1, 8), _F32).at[0, 0].set(params["out_layer"]["b"][0])
    y = _final_call(tabh, params["emb_out"]["W"],
                    params["emb_out"]["b"][None, :], wo, bo)
    return y[:_N, :1]
